# Initial kernel scaffold; baseline (speedup 1.0000x reference)
import jax
import jax.numpy as jnp
from jax.experimental import pallas as pl


def kernel(x, codebooks):
    raise NotImplementedError("write your pallas kernel here")


# speedup vs baseline: 2.9442x; 2.9442x over previous
import functools

import jax
import jax.numpy as jnp
from jax import lax
from jax.experimental import pallas as pl
from jax.experimental.pallas import tpu as pltpu
from jax.experimental.pallas import tpu_sc as plsc

_N, _D, _M, _K = 65536, 128, 8, 256
_DSUB = _D // _M
_BN = 1024
_NW = 32
_B = _N * _M
_BPW = _B // _NW
_CH = 4096


def _dist_body(x_ref, w_ref, codes_ref, psq_ref):
    xb = x_ref[...]
    w = w_ref[...]
    ip = jnp.dot(xb, w, preferred_element_type=jnp.float32)
    c2 = jnp.sum(w * w, axis=0, keepdims=True)
    scores = c2 - 2.0 * ip
    acc = jnp.sum(xb * xb, axis=1, keepdims=True)
    cols = []
    for m in range(_M):
        s = scores[:, m * _K:(m + 1) * _K]
        acc = acc + jnp.min(s, axis=1, keepdims=True)
        am = jnp.argmin(s, axis=1).astype(jnp.int32)
        cols.append(am[:, None] + (m * _K))
    codes_ref[...] = jnp.concatenate(cols, axis=1)
    psq_ref[...] = acc


def _loss_body(psq_ref, loss_ref):
    psq = psq_ref[...]
    mean_sq = jnp.sum(psq) / (_N * _D)
    loss_ref[...] = jnp.sqrt(jnp.maximum(psq, 0.0)) + mean_sq


@functools.cache
def _make_sc_gather():
    mesh = plsc.VectorSubcoreMesh(core_axis_name="c", subcore_axis_name="s")

    @functools.partial(
        pl.kernel,
        mesh=mesh,
        out_type=jax.ShapeDtypeStruct((_B, _DSUB), jnp.float32),
        scratch_types=[
            pltpu.VMEM((_CH,), jnp.int32),
            pltpu.VMEM((_CH, _DSUB), jnp.float32),
            pltpu.SemaphoreType.DMA,
        ],
        compiler_params=pltpu.CompilerParams(use_tc_tiling_on_sc=False),
    )
    def _sc_gather(table_hbm, idx_hbm, out_hbm, idx_v, rows_v, sem):
        wid = lax.axis_index("s") * 2 + lax.axis_index("c")
        base = wid * _BPW
        for j in range(_BPW // _CH):
            off = base + j * _CH
            pltpu.sync_copy(idx_hbm.at[pl.ds(off, _CH)], idx_v)
            pltpu.async_copy(table_hbm.at[idx_v], rows_v, sem).wait()
            pltpu.sync_copy(rows_v, out_hbm.at[pl.ds(off, _CH)])

    return _sc_gather


def kernel(x, codebooks):
    w = jnp.zeros((_D, _M * _K), dtype=jnp.float32)
    for m in range(_M):
        w = lax.dynamic_update_slice(
            w, codebooks[m].T, (m * _DSUB, m * _K))

    codes, psq = pl.pallas_call(
        _dist_body,
        grid=(_N // _BN,),
        in_specs=[
            pl.BlockSpec((_BN, _D), lambda i: (i, 0)),
            pl.BlockSpec((_D, _M * _K), lambda i: (0, 0)),
        ],
        out_specs=[
            pl.BlockSpec((_BN, _M), lambda i: (i, 0)),
            pl.BlockSpec((_BN, 1), lambda i: (i, 0)),
        ],
        out_shape=[
            jax.ShapeDtypeStruct((_N, _M), jnp.int32),
            jax.ShapeDtypeStruct((_N, 1), jnp.float32),
        ],
    )(x, w)

    table = codebooks.reshape(_M * _K, _DSUB)
    recon_rows = _make_sc_gather()(table, codes.reshape(_B))

    loss2d = pl.pallas_call(
        _loss_body,
        out_shape=jax.ShapeDtypeStruct((_N // 128, 128), jnp.float32),
    )(psq.reshape(_N // 128, 128))

    return recon_rows.reshape(_N, _D), loss2d.reshape(_N)
